# Initial kernel scaffold; baseline (speedup 1.0000x reference)
import jax
import jax.numpy as jnp
from jax.experimental import pallas as pl


def kernel(g, table):
    raise NotImplementedError("write your pallas kernel here")


# speedup vs baseline: 5.1861x; 5.1861x over previous
import functools

import jax
import jax.numpy as jnp
from jax import lax
from jax.experimental import pallas as pl
from jax.experimental.pallas import tpu as pltpu
from jax.experimental.pallas import tpu_sc as plsc

_NC = 2
_NS = 16
_NW = _NC * _NS
_L = 16

_B = 16384
_T = 200
_N = _B * _T
_PER_W = _N // _NW
_CHUNK = 10240
_STEPS = _PER_W // _CHUNK
_GROUPS = _CHUNK // _L
_TBL_PAD = 32


def _body(g_hbm, tbl_hbm, out_hbm, idx_v, out_v, tbl_v):
    wid = lax.axis_index("s") * _NC + lax.axis_index("c")
    base = wid * _PER_W
    pltpu.sync_copy(tbl_hbm, tbl_v)
    lane = lax.iota(jnp.int32, _L)
    soff = [lane * 4 + c for c in range(4)]

    def step(k, carry):
        off = base + k * _CHUNK
        pltpu.sync_copy(g_hbm.at[pl.ds(off, _CHUNK)], idx_v)

        def grp(i, c2):
            gvec = idx_v[pl.ds(i * _L, _L)]
            t4 = gvec * 4
            ob = i * (_L * 4)
            for c in range(4):
                val = plsc.load_gather(tbl_v, [t4 + c])
                plsc.store_scatter(out_v, [ob + soff[c]], val)
            return c2

        lax.fori_loop(0, _GROUPS, grp, 0)
        pltpu.sync_copy(out_v, out_hbm.at[pl.ds(off * 4, _CHUNK * 4)])
        return carry

    lax.fori_loop(0, _STEPS, step, 0)


@jax.jit
def kernel(g, table):
    tbl_flat = jnp.pad(table.reshape(-1), (0, _TBL_PAD - table.size))
    mesh = plsc.VectorSubcoreMesh(core_axis_name="c", subcore_axis_name="s")
    run = pl.kernel(
        _body,
        mesh=mesh,
        out_type=jax.ShapeDtypeStruct((_N * 4,), jnp.float32),
        scratch_types=[
            pltpu.VMEM((_CHUNK,), jnp.int32),
            pltpu.VMEM((_CHUNK * 4,), jnp.float32),
            pltpu.VMEM((_TBL_PAD,), jnp.float32),
        ],
        compiler_params=pltpu.CompilerParams(needs_layout_passes=False),
    )
    out_flat = run(g.reshape(-1), tbl_flat)
    return out_flat.reshape(_B, _T, 4)


# speedup vs baseline: 68.9274x; 13.2909x over previous
import functools

import jax
import jax.numpy as jnp
from jax import lax
from jax.experimental import pallas as pl
from jax.experimental.pallas import tpu as pltpu
from jax.experimental.pallas import tpu_sc as plsc

_NC = 2
_NS = 16
_NW = _NC * _NS
_L = 16

_B = 16384
_T = 200
_Q = 4
_BQ = _B // _Q
_TASKS = _T * _Q
_PER_W = _TASKS // _NW
_GROUPS = _BQ // _L
_TBL_PAD = 32


def _body(g_hbm, tbl_hbm, out_hbm, gv, ov, tv):
    wid = lax.axis_index("s") * _NC + lax.axis_index("c")
    task0 = wid * _PER_W
    pltpu.sync_copy(tbl_hbm, tv)

    def task(k, carry):
        tid = task0 + k
        t = tid // _Q
        q = tid % _Q
        b0 = q * _BQ
        pltpu.sync_copy(g_hbm.at[t, pl.ds(b0, _BQ)], gv)

        def grp(i, c2):
            gvec = gv[pl.ds(i * _L, _L)]
            t4 = gvec * 4
            for c in range(4):
                val = plsc.load_gather(tv, [t4 + c])
                ov[c, pl.ds(i * _L, _L)] = val
            return c2

        lax.fori_loop(0, _GROUPS, grp, 0, unroll=4)
        pltpu.sync_copy(ov, out_hbm.at[t, :, pl.ds(b0, _BQ)])
        return carry

    lax.fori_loop(0, _PER_W, task, 0)


@jax.jit
def kernel(g, table):
    tbl_flat = jnp.pad(table.reshape(-1), (0, _TBL_PAD - table.size))
    mesh = plsc.VectorSubcoreMesh(core_axis_name="c", subcore_axis_name="s")
    run = pl.kernel(
        _body,
        mesh=mesh,
        out_type=jax.ShapeDtypeStruct((_T, 4, _B), jnp.float32),
        scratch_types=[
            pltpu.VMEM((_BQ,), jnp.int32),
            pltpu.VMEM((4, _BQ), jnp.float32),
            pltpu.VMEM((_TBL_PAD,), jnp.float32),
        ],
        compiler_params=pltpu.CompilerParams(needs_layout_passes=False),
    )
    outP = run(g.T, tbl_flat)
    return outP.transpose(2, 0, 1)


# speedup vs baseline: 90.8985x; 1.3188x over previous
import functools

import jax
import jax.numpy as jnp
from jax import lax
from jax.experimental import pallas as pl
from jax.experimental.pallas import tpu as pltpu
from jax.experimental.pallas import tpu_sc as plsc

_NC = 2
_NS = 16
_NW = _NC * _NS
_L = 16

_B = 16384
_T = 200
_Q = 8
_BQ = _B // _Q
_TASKS = _T * _Q
_PER_W = _TASKS // _NW
_PAIRS = _PER_W // 2
_GROUPS = _BQ // _L
_CSTRIDE = 16


def _task_coords(tid):
    t = tid // _Q
    b0 = (tid % _Q) * _BQ
    return t, b0


def _body(g_hbm, tbl_hbm, out_hbm,
          gv0, gv1, ov0, ov1, tv0, tv1, tv2, tv3,
          isem0, isem1, osem0, osem1):
    wid = lax.axis_index("s") * _NC + lax.axis_index("c")
    task0 = wid * _PER_W
    gvs = (gv0, gv1)
    ovs = (ov0, ov1)
    isems = (isem0, isem1)
    osems = (osem0, osem1)
    for c, tv in enumerate((tv0, tv1, tv2, tv3)):
        pltpu.sync_copy(tbl_hbm.at[pl.ds(c * _CSTRIDE, _CSTRIDE)], tv)

    def in_copy(tid, buf):
        t, b0 = _task_coords(tid)
        return pltpu.make_async_copy(g_hbm.at[t, pl.ds(b0, _BQ)], gvs[buf],
                                     isems[buf])

    def out_copy(tid, buf):
        t, b0 = _task_coords(tid)
        return pltpu.make_async_copy(ovs[buf], out_hbm.at[t, :, pl.ds(b0, _BQ)],
                                     osems[buf])

    in_copy(task0, 0).start()

    def pair(j, carry):
        for buf in range(2):
            tid = task0 + j * 2 + buf
            in_copy(tid, buf).wait()
            if buf == 0:
                in_copy(tid + 1, 1).start()
            else:
                @pl.when(j < _PAIRS - 1)
                def _():
                    in_copy(tid + 1, 0).start()
            @pl.when(j > 0)
            def _():
                out_copy(tid, buf).wait()

            gv = gvs[buf]
            ov = ovs[buf]

            def grp(i, c2):
                gvec = gv[pl.ds(i * _L, _L)]
                for c, tv in enumerate((tv0, tv1, tv2, tv3)):
                    ov[c, pl.ds(i * _L, _L)] = plsc.load_gather(tv, [gvec])
                return c2

            lax.fori_loop(0, _GROUPS, grp, 0, unroll=4)
            out_copy(tid, buf).start()
        return carry

    lax.fori_loop(0, _PAIRS, pair, 0)
    out_copy(task0 + _PER_W - 2, 0).wait()
    out_copy(task0 + _PER_W - 1, 1).wait()


@jax.jit
def kernel(g, table):
    tblT = jnp.pad(table.T, ((0, 0), (0, _CSTRIDE - table.shape[0]))).reshape(-1)
    mesh = plsc.VectorSubcoreMesh(core_axis_name="c", subcore_axis_name="s")
    run = pl.kernel(
        _body,
        mesh=mesh,
        out_type=jax.ShapeDtypeStruct((_T, 4, _B), jnp.float32),
        scratch_types=[
            pltpu.VMEM((_BQ,), jnp.int32),
            pltpu.VMEM((_BQ,), jnp.int32),
            pltpu.VMEM((4, _BQ), jnp.float32),
            pltpu.VMEM((4, _BQ), jnp.float32),
            pltpu.VMEM((_CSTRIDE,), jnp.float32),
            pltpu.VMEM((_CSTRIDE,), jnp.float32),
            pltpu.VMEM((_CSTRIDE,), jnp.float32),
            pltpu.VMEM((_CSTRIDE,), jnp.float32),
            pltpu.SemaphoreType.DMA,
            pltpu.SemaphoreType.DMA,
            pltpu.SemaphoreType.DMA,
            pltpu.SemaphoreType.DMA,
        ],
        compiler_params=pltpu.CompilerParams(needs_layout_passes=False),
    )
    outP = run(g.T, tblT)
    return outP.transpose(2, 0, 1)


# speedup vs baseline: 191.3710x; 2.1053x over previous
import functools

import jax
import jax.numpy as jnp
from jax import lax
from jax.experimental import pallas as pl
from jax.experimental.pallas import tpu as pltpu
from jax.experimental.pallas import tpu_sc as plsc

_NC = 2
_NS = 16
_NW = _NC * _NS
_L = 16

_B = 16384
_T = 200
_Q = 8
_BQ = _B // _Q
_TASKS = _T * _Q
_PER_W = _TASKS // _NW
_PAIRS = _PER_W // 2
_GROUPS = _BQ // _L
_CSTRIDE = 16


def _task_coords(tid):
    t = tid // _Q
    b0 = (tid % _Q) * _BQ
    return t, b0


def _body(g_hbm, tbl_hbm, out_hbm,
          gv0, gv1, ov0, ov1, tv0, tv1, tv2, tv3,
          isem0, isem1, osem0, osem1):
    wid = lax.axis_index("s") * _NC + lax.axis_index("c")
    task0 = wid * _PER_W
    gvs = (gv0, gv1)
    ovs = (ov0, ov1)
    isems = (isem0, isem1)
    osems = (osem0, osem1)
    for c, tv in enumerate((tv0, tv1, tv2, tv3)):
        pltpu.sync_copy(tbl_hbm.at[pl.ds(c * _CSTRIDE, _CSTRIDE)], tv)

    def in_copy(tid, buf):
        t, b0 = _task_coords(tid)
        return pltpu.make_async_copy(g_hbm.at[t, pl.ds(b0, _BQ)], gvs[buf],
                                     isems[buf])

    def out_copy(tid, buf):
        t, b0 = _task_coords(tid)
        return pltpu.make_async_copy(ovs[buf], out_hbm.at[t, :, pl.ds(b0, _BQ)],
                                     osems[buf])

    in_copy(task0, 0).start()

    def pair(j, carry):
        for buf in range(2):
            tid = task0 + j * 2 + buf
            in_copy(tid, buf).wait()
            if buf == 0:
                in_copy(tid + 1, 1).start()
            else:
                @pl.when(j < _PAIRS - 1)
                def _():
                    in_copy(tid + 1, 0).start()
            @pl.when(j > 0)
            def _():
                out_copy(tid, buf).wait()

            gv = gvs[buf]
            ov = ovs[buf]

            @plsc.parallel_loop(0, _GROUPS, unroll=4)
            def grp(i):
                gvec = gv[pl.ds(i * _L, _L)]
                for c, tv in enumerate((tv0, tv1, tv2, tv3)):
                    ov[c, pl.ds(i * _L, _L)] = plsc.load_gather(tv, [gvec])

            out_copy(tid, buf).start()
        return carry

    lax.fori_loop(0, _PAIRS, pair, 0)
    out_copy(task0 + _PER_W - 2, 0).wait()
    out_copy(task0 + _PER_W - 1, 1).wait()


@jax.jit
def kernel(g, table):
    tblT = jnp.pad(table.T, ((0, 0), (0, _CSTRIDE - table.shape[0]))).reshape(-1)
    mesh = plsc.VectorSubcoreMesh(core_axis_name="c", subcore_axis_name="s")
    run = pl.kernel(
        _body,
        mesh=mesh,
        out_type=jax.ShapeDtypeStruct((_T, 4, _B), jnp.float32),
        scratch_types=[
            pltpu.VMEM((_BQ,), jnp.int32),
            pltpu.VMEM((_BQ,), jnp.int32),
            pltpu.VMEM((4, _BQ), jnp.float32),
            pltpu.VMEM((4, _BQ), jnp.float32),
            pltpu.VMEM((_CSTRIDE,), jnp.float32),
            pltpu.VMEM((_CSTRIDE,), jnp.float32),
            pltpu.VMEM((_CSTRIDE,), jnp.float32),
            pltpu.VMEM((_CSTRIDE,), jnp.float32),
            pltpu.SemaphoreType.DMA,
            pltpu.SemaphoreType.DMA,
            pltpu.SemaphoreType.DMA,
            pltpu.SemaphoreType.DMA,
        ],
        compiler_params=pltpu.CompilerParams(needs_layout_passes=False),
    )
    outP = run(g.T, tblT)
    return outP.transpose(2, 0, 1)


# speedup vs baseline: 242.3442x; 1.2664x over previous
import functools

import jax
import jax.numpy as jnp
from jax import lax
from jax.experimental import pallas as pl
from jax.experimental.pallas import tpu as pltpu
from jax.experimental.pallas import tpu_sc as plsc

_NC = 2
_NS = 16
_NW = _NC * _NS
_L = 16

_B = 16384
_T = 200
_Q = 4
_BQ = _B // _Q
_TASKS = _T * _Q
_PER_W = _TASKS // _NW
_PAIRS = (_PER_W - 1) // 2
_GROUPS = _BQ // _L
_CSTRIDE = 16


def _task_coords(tid):
    t = tid // _Q
    b0 = (tid % _Q) * _BQ
    return t, b0


def _body(g_hbm, tbl_hbm, out_hbm,
          gv0, gv1, ov0, ov1, tv0, tv1, tv2, tv3,
          isem0, isem1, osem0, osem1):
    wid = lax.axis_index("s") * _NC + lax.axis_index("c")
    task0 = wid * _PER_W
    tvs = (tv0, tv1, tv2, tv3)
    for c, tv in enumerate(tvs):
        pltpu.sync_copy(tbl_hbm.at[pl.ds(c * _CSTRIDE, _CSTRIDE)], tv)

    def in_copy(tid, gv, isem):
        t, b0 = _task_coords(tid)
        return pltpu.make_async_copy(g_hbm.at[t, pl.ds(b0, _BQ)], gv, isem)

    def out_copy(tid, ov, osem):
        t, b0 = _task_coords(tid)
        return pltpu.make_async_copy(ov, out_hbm.at[t, :, pl.ds(b0, _BQ)],
                                     osem)

    def compute(gv, ov):
        @plsc.parallel_loop(0, _GROUPS, unroll=4)
        def grp(i):
            gvec = gv[pl.ds(i * _L, _L)]
            for c, tv in enumerate(tvs):
                ov[c, pl.ds(i * _L, _L)] = plsc.load_gather(tv, [gvec])

    in_copy(task0, gv0, isem0).start()
    in_copy(task0, gv0, isem0).wait()
    in_copy(task0 + 1, gv1, isem1).start()
    compute(gv0, ov0)
    out_copy(task0, ov0, osem0).start()

    def pair(j, carry):
        t1 = task0 + 1 + 2 * j
        in_copy(t1, gv1, isem1).wait()
        in_copy(t1 + 1, gv0, isem0).start()

        @pl.when(j > 0)
        def _():
            out_copy(t1, ov1, osem1).wait()

        compute(gv1, ov1)
        out_copy(t1, ov1, osem1).start()

        t2 = t1 + 1
        in_copy(t2, gv0, isem0).wait()

        @pl.when(j < _PAIRS - 1)
        def _():
            in_copy(t2 + 1, gv1, isem1).start()

        out_copy(t2, ov0, osem0).wait()
        compute(gv0, ov0)
        out_copy(t2, ov0, osem0).start()
        return carry

    lax.fori_loop(0, _PAIRS, pair, 0)
    out_copy(task0 + _PER_W - 2, ov1, osem1).wait()
    out_copy(task0 + _PER_W - 1, ov0, osem0).wait()


@jax.jit
def kernel(g, table):
    tblT = jnp.pad(table.T, ((0, 0), (0, _CSTRIDE - table.shape[0]))).reshape(-1)
    mesh = plsc.VectorSubcoreMesh(core_axis_name="c", subcore_axis_name="s")
    run = pl.kernel(
        _body,
        mesh=mesh,
        out_type=jax.ShapeDtypeStruct((_T, 4, _B), jnp.float32),
        scratch_types=[
            pltpu.VMEM((_BQ,), jnp.int32),
            pltpu.VMEM((_BQ,), jnp.int32),
            pltpu.VMEM((4, _BQ), jnp.float32),
            pltpu.VMEM((4, _BQ), jnp.float32),
            pltpu.VMEM((_CSTRIDE,), jnp.float32),
            pltpu.VMEM((_CSTRIDE,), jnp.float32),
            pltpu.VMEM((_CSTRIDE,), jnp.float32),
            pltpu.VMEM((_CSTRIDE,), jnp.float32),
            pltpu.SemaphoreType.DMA,
            pltpu.SemaphoreType.DMA,
            pltpu.SemaphoreType.DMA,
            pltpu.SemaphoreType.DMA,
        ],
        compiler_params=pltpu.CompilerParams(needs_layout_passes=False),
    )
    outP = run(g.T, tblT)
    return outP.transpose(2, 0, 1)


# speedup vs baseline: 249.6007x; 1.0299x over previous
import functools

import jax
import jax.numpy as jnp
from jax import lax
from jax.experimental import pallas as pl
from jax.experimental.pallas import tpu as pltpu
from jax.experimental.pallas import tpu_sc as plsc

_NC = 2
_NS = 16
_NW = _NC * _NS
_L = 16

_B = 16384
_T = 200
_Q = 4
_BQ = _B // _Q
_TASKS = _T * _Q
_PER_W = _TASKS // _NW
_PAIRS = (_PER_W - 1) // 2
_GROUPS = _BQ // _L
_CSTRIDE = 16


def _task_coords(tid):
    t = tid // _Q
    b0 = (tid % _Q) * _BQ
    return t, b0


def _body(g_hbm, tbl_hbm, out_hbm,
          gv0, gv1, ov0, ov1, tv0,
          isem0, isem1, osem0, osem1):
    wid = lax.axis_index("s") * _NC + lax.axis_index("c")
    task0 = wid * _PER_W

    def in_copy(tid, gv, isem):
        t, b0 = _task_coords(tid)
        return pltpu.make_async_copy(g_hbm.at[t, pl.ds(b0, _BQ)], gv, isem)

    def out_copy(tid, ov, osem):
        t, b0 = _task_coords(tid)
        return pltpu.make_async_copy(ov, out_hbm.at[t, :, pl.ds(b0, _BQ)],
                                     osem)

    in_copy(task0, gv0, isem0).start()
    pltpu.sync_copy(tbl_hbm, tv0)

    def compute(gv, ov):
        @plsc.parallel_loop(0, _GROUPS, unroll=8)
        def grp(i):
            gvec = gv[pl.ds(i * _L, _L)]
            for c in range(4):
                idx = gvec if c == 0 else gvec + (c * _CSTRIDE)
                ov[c, pl.ds(i * _L, _L)] = plsc.load_gather(tv0, [idx])

    in_copy(task0, gv0, isem0).wait()
    in_copy(task0 + 1, gv1, isem1).start()
    compute(gv0, ov0)
    out_copy(task0, ov0, osem0).start()

    def pair(j, carry):
        t1 = task0 + 1 + 2 * j
        in_copy(t1, gv1, isem1).wait()
        in_copy(t1 + 1, gv0, isem0).start()

        @pl.when(j > 0)
        def _():
            out_copy(t1, ov1, osem1).wait()

        compute(gv1, ov1)
        out_copy(t1, ov1, osem1).start()

        t2 = t1 + 1
        in_copy(t2, gv0, isem0).wait()

        @pl.when(j < _PAIRS - 1)
        def _():
            in_copy(t2 + 1, gv1, isem1).start()

        out_copy(t2, ov0, osem0).wait()
        compute(gv0, ov0)
        out_copy(t2, ov0, osem0).start()
        return carry

    lax.fori_loop(0, _PAIRS, pair, 0)
    out_copy(task0 + _PER_W - 2, ov1, osem1).wait()
    out_copy(task0 + _PER_W - 1, ov0, osem0).wait()


@jax.jit
def kernel(g, table):
    tblT = jnp.pad(table.T, ((0, 0), (0, _CSTRIDE - table.shape[0]))).reshape(-1)
    mesh = plsc.VectorSubcoreMesh(core_axis_name="c", subcore_axis_name="s")
    run = pl.kernel(
        _body,
        mesh=mesh,
        out_type=jax.ShapeDtypeStruct((_T, 4, _B), jnp.float32),
        scratch_types=[
            pltpu.VMEM((_BQ,), jnp.int32),
            pltpu.VMEM((_BQ,), jnp.int32),
            pltpu.VMEM((4, _BQ), jnp.float32),
            pltpu.VMEM((4, _BQ), jnp.float32),
            pltpu.VMEM((4 * _CSTRIDE,), jnp.float32),
            pltpu.SemaphoreType.DMA,
            pltpu.SemaphoreType.DMA,
            pltpu.SemaphoreType.DMA,
            pltpu.SemaphoreType.DMA,
        ],
        compiler_params=pltpu.CompilerParams(needs_layout_passes=False),
    )
    outP = run(g.T, tblT)
    return outP.transpose(2, 0, 1)
